# baseline (speedup 1.0000x reference)
import functools

import jax
import jax.numpy as jnp
import numpy as np
from jax import lax
from jax.experimental import pallas as pl
from jax.experimental.pallas import tpu as pltpu
from jax.experimental.pallas import tpu_sc as plsc

B = 1024
N = 8192
D = 3
K = 12
H = 512
SHELL = 12
EV = 2
FLAT = K * D
OUT = SHELL * D * EV
NF = N * D
FP = 48

NC = 2
NS = 16
NW = NC * NS
RPW = B // NW
STEPS = N // 16


def _sc_knn_body(sol_hbm, ref_hbm, out_hbm,
                 xb0, yb0, zb0, xb1, yb1, zb1, refbuf, dbuf, chunklist,
                 cand_d2, cand_idx2, winbuf, outbuf, sem0, sem1):
    cid = lax.axis_index("c")
    sid = lax.axis_index("s")
    wid = sid * NC + cid
    base = wid * RPW

    iota = lax.broadcasted_iota(jnp.int32, (16,), 0)
    iota3 = iota * 3
    inf_v = jnp.full((16,), jnp.float32(np.inf), jnp.float32)
    zero_v = jnp.zeros((16,), jnp.float32)
    mask12 = iota < 12

    pltpu.sync_copy(ref_hbm.at[pl.ds(base * 3, RPW * 3)],
                    refbuf.at[pl.ds(0, RPW * 3)])

    def _zero(i, c):
        outbuf[pl.ds(i * 16, 16)] = zero_v
        return c
    lax.fori_loop(0, RPW * FP // 16, _zero, 0)

    def _fetch(r, bufs, sem):
        xb, yb, zb = bufs
        pltpu.async_copy(sol_hbm.at[0, r], xb, sem)
        pltpu.async_copy(sol_hbm.at[1, r], yb, sem)
        pltpu.async_copy(sol_hbm.at[2, r], zb, sem)

    def _wait(r, bufs, sem):
        xb, yb, zb = bufs
        pltpu.make_async_copy(sol_hbm.at[0, r], xb, sem).wait()
        pltpu.make_async_copy(sol_hbm.at[1, r], yb, sem).wait()
        pltpu.make_async_copy(sol_hbm.at[2, r], zb, sem).wait()

    bufs0 = (xb0, yb0, zb0)
    bufs1 = (xb1, yb1, zb1)

    _fetch(base, bufs0, sem0)

    def _row(j, bufs, sem_cur, bufs_next, sem_next):
        r = base + j
        xb, yb, zb = bufs
        rn = jnp.minimum(r + 1, B - 1)
        _fetch(rn, bufs_next, sem_next)
        _wait(r, bufs, sem_cur)

        jb = j * 3
        rx = plsc.load_gather(refbuf, [jnp.full((16,), jb, jnp.int32)])
        ry = plsc.load_gather(refbuf, [jnp.full((16,), jb + 1, jnp.int32)])
        rz = plsc.load_gather(refbuf, [jnp.full((16,), jb + 2, jnp.int32)])

        def _dist(s, rmin):
            o = s * 16
            dx = xb[pl.ds(o, 16)] - rx
            dy = yb[pl.ds(o, 16)] - ry
            dz = zb[pl.ds(o, 16)] - rz
            d = (dx * dx + dy * dy) + dz * dz
            dbuf[pl.ds(o, 16)] = d
            return jnp.minimum(rmin, d)
        rmin = plsc.parallel_loop(0, STEPS, carry=inf_v, unroll=16)(_dist)
        thresh = jnp.max(rmin)

        lane0 = iota == 0
        def _compress(s, nsl):
            d = dbuf[pl.ds(s * 16, 16)]
            msk = d <= thresh
            pc = plsc.all_reduce_population_count(msk)
            qual = pc > 0
            plsc.store_scatter(chunklist, [nsl],
                               jnp.full((16,), s, jnp.int32),
                               mask=lane0 & qual)
            return nsl + jnp.where(qual, 1, 0)
        nsl_v = plsc.parallel_loop(0, STEPS,
                                   carry=jnp.zeros((16,), jnp.int32),
                                   unroll=16)(_compress)
        nslots = jnp.max(nsl_v)

        def _pack(c, cnt):
            sv = plsc.load_gather(chunklist, [jnp.full((16,), c, jnp.int32)])
            ii = sv * 16 + iota
            d = plsc.load_gather(dbuf, [ii])
            msk = d <= thresh
            pos = cnt + plsc.cumsum(jnp.where(msk, 1, 0)) - 1
            plsc.store_scatter(cand_d2, [pos], d, mask=msk)
            plsc.store_scatter(cand_idx2, [pos], ii, mask=msk)
            return cnt + plsc.all_reduce_population_count(msk)
        cnt2 = plsc.parallel_loop(0, nslots,
                                  carry=jnp.zeros((16,), jnp.int32))(_pack)
        cnt2s = jnp.max(cnt2)
        plsc.store_scatter(cand_d2, [cnt2s + iota], inf_v)
        nch = cnt2s // 16 + 1

        winbuf[pl.ds(0, 16)] = jnp.zeros((16,), jnp.int32)

        def _round(k, c2):
            def _scan(c, carry):
                bd, bi, bp = carry
                d = cand_d2[pl.ds(c * 16, 16)]
                ii = cand_idx2[pl.ds(c * 16, 16)]
                pp = c * 16 + iota
                better = (d < bd) | ((d == bd) & (ii < bi))
                return (jnp.where(better, d, bd),
                        jnp.where(better, ii, bi),
                        jnp.where(better, pp, bp))
            big_i = jnp.full((16,), jnp.int32(2 ** 30), jnp.int32)
            bd, bi, bp = plsc.parallel_loop(
                0, nch, carry=(inf_v, big_i, big_i))(_scan)
            m = jnp.min(bd)
            eq = bd == m
            w = jnp.min(jnp.where(eq, bi, big_i))
            p = jnp.min(jnp.where(eq & (bi == w), bp, big_i))
            plsc.store_scatter(winbuf, [iota], jnp.full((16,), w, jnp.int32),
                               mask=iota == k)
            plsc.store_scatter(cand_d2, [jnp.full((16,), p, jnp.int32)], inf_v,
                               mask=iota == 0)
            return c2
        lax.fori_loop(0, K, _round, 0)

        wi = winbuf[pl.ds(0, 16)]
        xs = plsc.load_gather(xb, [wi]) - rx
        ys = plsc.load_gather(yb, [wi]) - ry
        zs = plsc.load_gather(zb, [wi]) - rz
        po = j * FP + iota3
        plsc.store_scatter(outbuf, [po], xs, mask=mask12)
        plsc.store_scatter(outbuf, [po + 1], ys, mask=mask12)
        plsc.store_scatter(outbuf, [po + 2], zs, mask=mask12)

    def _pair(t, c):
        _row(2 * t, bufs0, sem0, bufs1, sem1)
        _row(2 * t + 1, bufs1, sem1, bufs0, sem0)
        return c
    lax.fori_loop(0, RPW // 2, _pair, 0)

    _wait(0, bufs0, sem0)

    pltpu.sync_copy(outbuf, out_hbm.at[pl.ds(base * FP, RPW * FP)])


def _mlp_body(flat_ref, W1_ref, b1_ref, W2_ref, b2_ref, W3_ref, b3_ref,
              Wp_ref, bp_ref, Ws_ref, bs_ref, params_ref, shifts_ref):
    f = flat_ref[...]
    h = jnp.maximum(jnp.dot(f, W1_ref[...]) + b1_ref[...], 0.0)
    h = jnp.maximum(jnp.dot(h, W2_ref[...]) + b2_ref[...], 0.0)
    h = jnp.maximum(jnp.dot(h, W3_ref[...]) + b3_ref[...], 0.0)
    params_ref[...] = jnp.dot(h, Wp_ref[...]) + bp_ref[...]
    shifts_ref[...] = jnp.dot(f, Ws_ref[...]) + bs_ref[...]


@functools.partial(jax.jit, static_argnames=("interpret",))
def kernel(ref_coord, solute_coords, W1, b1, W2, b2, W3, b3, Wp, bp, Ws, bs,
           interpret=False):
    sol_t = jnp.transpose(solute_coords, (2, 0, 1))
    ref_flat = ref_coord.reshape(B * D)

    mesh = plsc.VectorSubcoreMesh(core_axis_name="c", subcore_axis_name="s",
                                  num_cores=NC, num_subcores=NS)
    sc_knn = pl.kernel(
        _sc_knn_body,
        out_type=jax.ShapeDtypeStruct((B * FP,), jnp.float32),
        mesh=mesh,
        scratch_types=[
            pltpu.VMEM((N,), jnp.float32),
            pltpu.VMEM((N,), jnp.float32),
            pltpu.VMEM((N,), jnp.float32),
            pltpu.VMEM((N,), jnp.float32),
            pltpu.VMEM((N,), jnp.float32),
            pltpu.VMEM((N,), jnp.float32),
            pltpu.VMEM((128,), jnp.float32),
            pltpu.VMEM((N,), jnp.float32),
            pltpu.VMEM((1024,), jnp.int32),
            pltpu.VMEM((N + 128,), jnp.float32),
            pltpu.VMEM((N + 128,), jnp.int32),
            pltpu.VMEM((128,), jnp.int32),
            pltpu.VMEM((RPW * FP,), jnp.float32),
            pltpu.SemaphoreType.DMA,
            pltpu.SemaphoreType.DMA,
        ],
        compiler_params=pltpu.CompilerParams(needs_layout_passes=False),
        interpret=interpret,
    )
    flat48 = sc_knn(sol_t, ref_flat).reshape(B, FP)

    W1p = jnp.pad(W1, ((0, FP - FLAT), (0, 0)))
    Wsp = jnp.pad(Ws, ((0, FP - FLAT), (0, 0)))

    params, shifts = pl.pallas_call(
        _mlp_body,
        out_shape=(
            jax.ShapeDtypeStruct((B, OUT), jnp.float32),
            jax.ShapeDtypeStruct((B, OUT), jnp.float32),
        ),
        interpret=interpret,
    )(flat48, W1p, b1.reshape(1, H), W2, b2.reshape(1, H), W3,
      b3.reshape(1, H), Wp, bp.reshape(1, OUT), Wsp, bs.reshape(1, OUT))
    return (params.reshape(B, SHELL, D, EV), shifts.reshape(B, SHELL, D, EV))


# speedup vs baseline: 1.0677x; 1.0677x over previous
import functools

import jax
import jax.numpy as jnp
import numpy as np
from jax import lax
from jax.experimental import pallas as pl
from jax.experimental.pallas import tpu as pltpu
from jax.experimental.pallas import tpu_sc as plsc

B = 1024
N = 8192
D = 3
K = 12
H = 512
SHELL = 12
EV = 2
FLAT = K * D
OUT = SHELL * D * EV
NF = N * D
FP = 48

NC = 2
NS = 16
NW = NC * NS
RPW = B // NW
RPW_H = (B // 2) // NW
STEPS = N // 16


def _sc_knn_body(sol_hbm, ref_hbm, out_hbm,
                 xb0, yb0, zb0, xb1, yb1, zb1, refbuf, dbuf, chunklist,
                 cand_d2, cand_idx2, winbuf, outbuf, sem0, sem1, *, half):
    cid = lax.axis_index("c")
    sid = lax.axis_index("s")
    wid = sid * NC + cid
    base = half * (B // 2) + wid * RPW_H

    iota = lax.broadcasted_iota(jnp.int32, (16,), 0)
    iota3 = iota * 3
    inf_v = jnp.full((16,), jnp.float32(np.inf), jnp.float32)
    zero_v = jnp.zeros((16,), jnp.float32)
    mask12 = iota < 12

    pltpu.sync_copy(ref_hbm.at[pl.ds(base * 3, RPW_H * 3)],
                    refbuf.at[pl.ds(0, RPW_H * 3)])

    def _zero(i, c):
        outbuf[pl.ds(i * 16, 16)] = zero_v
        return c
    lax.fori_loop(0, RPW_H * FP // 16, _zero, 0)

    def _fetch(r, bufs, sem):
        xb, yb, zb = bufs
        pltpu.async_copy(sol_hbm.at[0, r], xb, sem)
        pltpu.async_copy(sol_hbm.at[1, r], yb, sem)
        pltpu.async_copy(sol_hbm.at[2, r], zb, sem)

    def _wait(r, bufs, sem):
        xb, yb, zb = bufs
        pltpu.make_async_copy(sol_hbm.at[0, r], xb, sem).wait()
        pltpu.make_async_copy(sol_hbm.at[1, r], yb, sem).wait()
        pltpu.make_async_copy(sol_hbm.at[2, r], zb, sem).wait()

    bufs0 = (xb0, yb0, zb0)
    bufs1 = (xb1, yb1, zb1)

    _fetch(base, bufs0, sem0)

    def _row(j, bufs, sem_cur, bufs_next, sem_next):
        r = base + j
        xb, yb, zb = bufs
        rn = jnp.minimum(r + 1, B - 1)
        _fetch(rn, bufs_next, sem_next)
        _wait(r, bufs, sem_cur)

        jb = j * 3
        rx = plsc.load_gather(refbuf, [jnp.full((16,), jb, jnp.int32)])
        ry = plsc.load_gather(refbuf, [jnp.full((16,), jb + 1, jnp.int32)])
        rz = plsc.load_gather(refbuf, [jnp.full((16,), jb + 2, jnp.int32)])

        def _dist(s, rmin):
            o = s * 16
            dx = xb[pl.ds(o, 16)] - rx
            dy = yb[pl.ds(o, 16)] - ry
            dz = zb[pl.ds(o, 16)] - rz
            d = (dx * dx + dy * dy) + dz * dz
            dbuf[pl.ds(o, 16)] = d
            return jnp.minimum(rmin, d)
        rmin = plsc.parallel_loop(0, STEPS, carry=inf_v, unroll=8)(_dist)
        thresh = jnp.max(rmin)

        lane0 = iota == 0
        def _compress(s, nsl):
            d = dbuf[pl.ds(s * 16, 16)]
            msk = d <= thresh
            pc = plsc.all_reduce_population_count(msk)
            qual = pc > 0
            plsc.store_scatter(chunklist, [nsl],
                               jnp.full((16,), s, jnp.int32),
                               mask=lane0 & qual)
            return nsl + jnp.where(qual, 1, 0)
        nsl_v = plsc.parallel_loop(0, STEPS,
                                   carry=jnp.zeros((16,), jnp.int32),
                                   unroll=8)(_compress)
        nslots = jnp.max(nsl_v)

        def _pack(c, cnt):
            sv = plsc.load_gather(chunklist, [jnp.full((16,), c, jnp.int32)])
            ii = sv * 16 + iota
            d = plsc.load_gather(dbuf, [ii])
            msk = d <= thresh
            pos = cnt + plsc.cumsum(jnp.where(msk, 1, 0)) - 1
            plsc.store_scatter(cand_d2, [pos], d, mask=msk)
            plsc.store_scatter(cand_idx2, [pos], ii, mask=msk)
            return cnt + plsc.all_reduce_population_count(msk)
        cnt2 = plsc.parallel_loop(0, nslots,
                                  carry=jnp.zeros((16,), jnp.int32))(_pack)
        cnt2s = jnp.max(cnt2)
        plsc.store_scatter(cand_d2, [cnt2s + iota], inf_v)
        nch = cnt2s // 16 + 1

        winbuf[pl.ds(0, 16)] = jnp.zeros((16,), jnp.int32)

        def _round(k, c2):
            def _scan(c, carry):
                bd, bi, bp = carry
                d = cand_d2[pl.ds(c * 16, 16)]
                ii = cand_idx2[pl.ds(c * 16, 16)]
                pp = c * 16 + iota
                better = (d < bd) | ((d == bd) & (ii < bi))
                return (jnp.where(better, d, bd),
                        jnp.where(better, ii, bi),
                        jnp.where(better, pp, bp))
            big_i = jnp.full((16,), jnp.int32(2 ** 30), jnp.int32)
            bd, bi, bp = plsc.parallel_loop(
                0, nch, carry=(inf_v, big_i, big_i))(_scan)
            m = jnp.min(bd)
            eq = bd == m
            w = jnp.min(jnp.where(eq, bi, big_i))
            p = jnp.min(jnp.where(eq & (bi == w), bp, big_i))
            plsc.store_scatter(winbuf, [iota], jnp.full((16,), w, jnp.int32),
                               mask=iota == k)
            plsc.store_scatter(cand_d2, [jnp.full((16,), p, jnp.int32)], inf_v,
                               mask=iota == 0)
            return c2
        lax.fori_loop(0, K, _round, 0)

        wi = winbuf[pl.ds(0, 16)]
        xs = plsc.load_gather(xb, [wi]) - rx
        ys = plsc.load_gather(yb, [wi]) - ry
        zs = plsc.load_gather(zb, [wi]) - rz
        po = j * FP + iota3
        plsc.store_scatter(outbuf, [po], xs, mask=mask12)
        plsc.store_scatter(outbuf, [po + 1], ys, mask=mask12)
        plsc.store_scatter(outbuf, [po + 2], zs, mask=mask12)

    def _pair(t, c):
        _row(2 * t, bufs0, sem0, bufs1, sem1)
        _row(2 * t + 1, bufs1, sem1, bufs0, sem0)
        return c
    lax.fori_loop(0, RPW_H // 2, _pair, 0)

    _wait(0, bufs0, sem0)

    pltpu.sync_copy(
        outbuf,
        out_hbm.at[pl.ds((base - half * (B // 2)) * FP, RPW_H * FP)])


def _mlp_body(flat_ref, W1_ref, b1_ref, W2_ref, b2_ref, W3_ref, b3_ref,
              Wp_ref, bp_ref, Ws_ref, bs_ref, params_ref, shifts_ref):
    f = flat_ref[...]
    h = jnp.maximum(jnp.dot(f, W1_ref[...]) + b1_ref[...], 0.0)
    h = jnp.maximum(jnp.dot(h, W2_ref[...]) + b2_ref[...], 0.0)
    h = jnp.maximum(jnp.dot(h, W3_ref[...]) + b3_ref[...], 0.0)
    params_ref[...] = jnp.dot(h, Wp_ref[...]) + bp_ref[...]
    shifts_ref[...] = jnp.dot(f, Ws_ref[...]) + bs_ref[...]


@functools.partial(jax.jit, static_argnames=("interpret",))
def kernel(ref_coord, solute_coords, W1, b1, W2, b2, W3, b3, Wp, bp, Ws, bs,
           interpret=False):
    sol_t = jnp.transpose(solute_coords, (2, 0, 1))
    ref_flat = ref_coord.reshape(B * D)

    mesh = plsc.VectorSubcoreMesh(core_axis_name="c", subcore_axis_name="s",
                                  num_cores=NC, num_subcores=NS)
    def make_knn(half):
        return pl.kernel(
        functools.partial(_sc_knn_body, half=half),
        out_type=jax.ShapeDtypeStruct((B // 2 * FP,), jnp.float32),
        mesh=mesh,
        scratch_types=[
            pltpu.VMEM((N,), jnp.float32),
            pltpu.VMEM((N,), jnp.float32),
            pltpu.VMEM((N,), jnp.float32),
            pltpu.VMEM((N,), jnp.float32),
            pltpu.VMEM((N,), jnp.float32),
            pltpu.VMEM((N,), jnp.float32),
            pltpu.VMEM((128,), jnp.float32),
            pltpu.VMEM((N,), jnp.float32),
            pltpu.VMEM((1024,), jnp.int32),
            pltpu.VMEM((N + 128,), jnp.float32),
            pltpu.VMEM((N + 128,), jnp.int32),
            pltpu.VMEM((128,), jnp.int32),
            pltpu.VMEM((RPW_H * FP,), jnp.float32),
            pltpu.SemaphoreType.DMA,
            pltpu.SemaphoreType.DMA,
        ],
        compiler_params=pltpu.CompilerParams(needs_layout_passes=False),
        interpret=interpret,
    )
    flat0 = make_knn(0)(sol_t, ref_flat).reshape(B // 2, FP)
    flat1 = make_knn(1)(sol_t, ref_flat).reshape(B // 2, FP)

    W1p = jnp.pad(W1, ((0, FP - FLAT), (0, 0)))
    Wsp = jnp.pad(Ws, ((0, FP - FLAT), (0, 0)))

    def mlp(flat48):
        return pl.pallas_call(
            _mlp_body,
            out_shape=(
                jax.ShapeDtypeStruct((B // 2, OUT), jnp.float32),
                jax.ShapeDtypeStruct((B // 2, OUT), jnp.float32),
            ),
            interpret=interpret,
        )(flat48, W1p, b1.reshape(1, H), W2, b2.reshape(1, H), W3,
          b3.reshape(1, H), Wp, bp.reshape(1, OUT), Wsp, bs.reshape(1, OUT))
    p0, s0 = mlp(flat0)
    p1, s1 = mlp(flat1)
    params = jnp.concatenate([p0, p1], axis=0)
    shifts = jnp.concatenate([s0, s1], axis=0)
    return (params.reshape(B, SHELL, D, EV), shifts.reshape(B, SHELL, D, EV))


# speedup vs baseline: 1.1236x; 1.0524x over previous
import functools

import jax
import jax.numpy as jnp
import numpy as np
from jax import lax
from jax.experimental import pallas as pl
from jax.experimental.pallas import tpu as pltpu
from jax.experimental.pallas import tpu_sc as plsc

B = 1024
N = 8192
D = 3
K = 12
H = 512
SHELL = 12
EV = 2
FLAT = K * D
OUT = SHELL * D * EV
NF = N * D
FP = 48

NC = 2
NS = 16
NW = NC * NS
RPW = B // NW
STEPS = N // 16


def _sc_knn_body(sol_hbm, ref_hbm, out_hbm,
                 xb0, yb0, zb0, xb1, yb1, zb1, refbuf, dbuf, chunklist,
                 cand_d2, cand_idx2, winbuf, outbuf, sem0, sem1):
    cid = lax.axis_index("c")
    sid = lax.axis_index("s")
    wid = sid * NC + cid
    base = wid * RPW

    iota = lax.broadcasted_iota(jnp.int32, (16,), 0)
    iota3 = iota * 3
    inf_v = jnp.full((16,), jnp.float32(np.inf), jnp.float32)
    zero_v = jnp.zeros((16,), jnp.float32)
    mask12 = iota < 12

    pltpu.sync_copy(ref_hbm.at[pl.ds(base * 3, RPW * 3)],
                    refbuf.at[pl.ds(0, RPW * 3)])

    def _zero(i, c):
        outbuf[pl.ds(i * 16, 16)] = zero_v
        return c
    lax.fori_loop(0, RPW * FP // 16, _zero, 0)

    def _fetch(r, bufs, sem):
        xb, yb, zb = bufs
        pltpu.async_copy(sol_hbm.at[0, r], xb, sem)
        pltpu.async_copy(sol_hbm.at[1, r], yb, sem)
        pltpu.async_copy(sol_hbm.at[2, r], zb, sem)

    def _wait(r, bufs, sem):
        xb, yb, zb = bufs
        pltpu.make_async_copy(sol_hbm.at[0, r], xb, sem).wait()
        pltpu.make_async_copy(sol_hbm.at[1, r], yb, sem).wait()
        pltpu.make_async_copy(sol_hbm.at[2, r], zb, sem).wait()

    bufs0 = (xb0, yb0, zb0)
    bufs1 = (xb1, yb1, zb1)

    _fetch(base, bufs0, sem0)

    def _row(j, bufs, sem_cur, bufs_next, sem_next):
        r = base + j
        xb, yb, zb = bufs
        rn = jnp.minimum(r + 1, B - 1)
        _fetch(rn, bufs_next, sem_next)
        _wait(r, bufs, sem_cur)

        jb = j * 3
        rx = plsc.load_gather(refbuf, [jnp.full((16,), jb, jnp.int32)])
        ry = plsc.load_gather(refbuf, [jnp.full((16,), jb + 1, jnp.int32)])
        rz = plsc.load_gather(refbuf, [jnp.full((16,), jb + 2, jnp.int32)])

        def _dist(s, rmin):
            o = s * 16
            dx = xb[pl.ds(o, 16)] - rx
            dy = yb[pl.ds(o, 16)] - ry
            dz = zb[pl.ds(o, 16)] - rz
            d = (dx * dx + dy * dy) + dz * dz
            dbuf[pl.ds(o, 16)] = d
            return jnp.minimum(rmin, d)
        rmin = plsc.parallel_loop(0, STEPS, carry=inf_v, unroll=8)(_dist)
        thresh = jnp.max(rmin)

        lane0 = iota == 0
        def _compress(s, nsl):
            d = dbuf[pl.ds(s * 16, 16)]
            msk = d <= thresh
            pc = plsc.all_reduce_population_count(msk)
            qual = pc > 0
            plsc.store_scatter(chunklist, [nsl],
                               jnp.full((16,), s, jnp.int32),
                               mask=lane0 & qual)
            return nsl + jnp.where(qual, 1, 0)
        nsl_v = plsc.parallel_loop(0, STEPS,
                                   carry=jnp.zeros((16,), jnp.int32),
                                   unroll=8)(_compress)
        nslots = jnp.max(nsl_v)

        def _pack(c, cnt):
            sv = plsc.load_gather(chunklist, [jnp.full((16,), c, jnp.int32)])
            ii = sv * 16 + iota
            d = plsc.load_gather(dbuf, [ii])
            msk = d <= thresh
            pos = cnt + plsc.cumsum(jnp.where(msk, 1, 0)) - 1
            plsc.store_scatter(cand_d2, [pos], d, mask=msk)
            plsc.store_scatter(cand_idx2, [pos], ii, mask=msk)
            return cnt + plsc.all_reduce_population_count(msk)
        cnt2 = plsc.parallel_loop(0, nslots,
                                  carry=jnp.zeros((16,), jnp.int32))(_pack)
        cnt2s = jnp.max(cnt2)
        plsc.store_scatter(cand_d2, [cnt2s + iota], inf_v)
        nch = cnt2s // 16 + 1

        winbuf[pl.ds(0, 16)] = jnp.zeros((16,), jnp.int32)

        def _round(k, c2):
            def _scan(c, carry):
                bd, bi, bp = carry
                d = cand_d2[pl.ds(c * 16, 16)]
                ii = cand_idx2[pl.ds(c * 16, 16)]
                pp = c * 16 + iota
                better = (d < bd) | ((d == bd) & (ii < bi))
                return (jnp.where(better, d, bd),
                        jnp.where(better, ii, bi),
                        jnp.where(better, pp, bp))
            big_i = jnp.full((16,), jnp.int32(2 ** 30), jnp.int32)
            bd, bi, bp = plsc.parallel_loop(
                0, nch, carry=(inf_v, big_i, big_i))(_scan)
            m = jnp.min(bd)
            eq = bd == m
            w = jnp.min(jnp.where(eq, bi, big_i))
            p = jnp.min(jnp.where(eq & (bi == w), bp, big_i))
            plsc.store_scatter(winbuf, [iota], jnp.full((16,), w, jnp.int32),
                               mask=iota == k)
            plsc.store_scatter(cand_d2, [jnp.full((16,), p, jnp.int32)], inf_v,
                               mask=iota == 0)
            return c2
        lax.fori_loop(0, K, _round, 0)

        wi = winbuf[pl.ds(0, 16)]
        xs = plsc.load_gather(xb, [wi]) - rx
        ys = plsc.load_gather(yb, [wi]) - ry
        zs = plsc.load_gather(zb, [wi]) - rz
        po = j * FP + iota3
        plsc.store_scatter(outbuf, [po], xs, mask=mask12)
        plsc.store_scatter(outbuf, [po + 1], ys, mask=mask12)
        plsc.store_scatter(outbuf, [po + 2], zs, mask=mask12)

    def _pair(t, c):
        _row(2 * t, bufs0, sem0, bufs1, sem1)
        _row(2 * t + 1, bufs1, sem1, bufs0, sem0)
        return c
    lax.fori_loop(0, RPW // 2, _pair, 0)

    _wait(0, bufs0, sem0)

    pltpu.sync_copy(outbuf, out_hbm.at[pl.ds(base * FP, RPW * FP)])


def _mlp_body(flat_ref, W1_ref, b1_ref, W2_ref, b2_ref, W3_ref, b3_ref,
              Wp_ref, bp_ref, Ws_ref, bs_ref, params_ref, shifts_ref):
    f = flat_ref[...]
    h = jnp.maximum(jnp.dot(f, W1_ref[...]) + b1_ref[...], 0.0)
    h = jnp.maximum(jnp.dot(h, W2_ref[...]) + b2_ref[...], 0.0)
    h = jnp.maximum(jnp.dot(h, W3_ref[...]) + b3_ref[...], 0.0)
    params_ref[...] = jnp.dot(h, Wp_ref[...]) + bp_ref[...]
    shifts_ref[...] = jnp.dot(f, Ws_ref[...]) + bs_ref[...]


@functools.partial(jax.jit, static_argnames=("interpret",))
def kernel(ref_coord, solute_coords, W1, b1, W2, b2, W3, b3, Wp, bp, Ws, bs,
           interpret=False):
    sol_t = jnp.transpose(solute_coords, (2, 0, 1))
    ref_flat = ref_coord.reshape(B * D)

    mesh = plsc.VectorSubcoreMesh(core_axis_name="c", subcore_axis_name="s",
                                  num_cores=NC, num_subcores=NS)
    sc_knn = pl.kernel(
        _sc_knn_body,
        out_type=jax.ShapeDtypeStruct((B * FP,), jnp.float32),
        mesh=mesh,
        scratch_types=[
            pltpu.VMEM((N,), jnp.float32),
            pltpu.VMEM((N,), jnp.float32),
            pltpu.VMEM((N,), jnp.float32),
            pltpu.VMEM((N,), jnp.float32),
            pltpu.VMEM((N,), jnp.float32),
            pltpu.VMEM((N,), jnp.float32),
            pltpu.VMEM((128,), jnp.float32),
            pltpu.VMEM((N,), jnp.float32),
            pltpu.VMEM((1024,), jnp.int32),
            pltpu.VMEM((N + 128,), jnp.float32),
            pltpu.VMEM((N + 128,), jnp.int32),
            pltpu.VMEM((128,), jnp.int32),
            pltpu.VMEM((RPW * FP,), jnp.float32),
            pltpu.SemaphoreType.DMA,
            pltpu.SemaphoreType.DMA,
        ],
        compiler_params=pltpu.CompilerParams(needs_layout_passes=False),
        interpret=interpret,
    )
    flat48 = sc_knn(sol_t, ref_flat).reshape(B, FP)

    W1p = jnp.pad(W1, ((0, FP - FLAT), (0, 0)))
    Wsp = jnp.pad(Ws, ((0, FP - FLAT), (0, 0)))

    params, shifts = pl.pallas_call(
        _mlp_body,
        out_shape=(
            jax.ShapeDtypeStruct((B, OUT), jnp.float32),
            jax.ShapeDtypeStruct((B, OUT), jnp.float32),
        ),
        interpret=interpret,
    )(flat48, W1p, b1.reshape(1, H), W2, b2.reshape(1, H), W3,
      b3.reshape(1, H), Wp, bp.reshape(1, OUT), Wsp, bs.reshape(1, OUT))
    return (params.reshape(B, SHELL, D, EV), shifts.reshape(B, SHELL, D, EV))
